# Initial kernel scaffold; baseline (speedup 1.0000x reference)
import jax
import jax.numpy as jnp
from jax.experimental import pallas as pl


def kernel(hidden_states, weight, e_score_correction_bias):
    raise NotImplementedError("write your pallas kernel here")


# speedup vs baseline: 1.9990x; 1.9990x over previous
import jax
import jax.numpy as jnp
from jax.experimental import pallas as pl

TOP_K = 8
N_EXPERTS = 64
N_GROUP = 8
GROUP_SIZE = 8
TOPK_GROUP = 4
SCALE = 2.5


def _router_block(x_ref, wt_ref, bias_ref, logits_ref, idx_ref, w_ref):
    x = x_ref[...]
    wt = wt_ref[...]
    logits = jnp.dot(x, wt, preferred_element_type=jnp.float32)
    logits_ref[...] = logits
    scores = jax.nn.sigmoid(logits)
    s4c = scores + bias_ref[...]

    bt = x.shape[0]
    lane = jax.lax.broadcasted_iota(jnp.int32, (bt, N_EXPERTS), 1)
    gid = lane // GROUP_SIZE
    neg = jnp.float32(-jnp.inf)

    gs_list = []
    for g in range(N_GROUP):
        mg = jnp.where(gid == g, s4c, neg)
        m1 = jnp.max(mg, axis=1, keepdims=True)
        first = jnp.min(jnp.where(mg == m1, lane, N_EXPERTS), axis=1,
                        keepdims=True)
        m2 = jnp.max(jnp.where(lane == first, neg, mg), axis=1, keepdims=True)
        gs_list.append(m1 + m2)
    group_scores = jnp.concatenate(gs_list, axis=1)

    glane = jax.lax.broadcasted_iota(jnp.int32, (bt, N_GROUP), 1)
    mask64 = jnp.zeros((bt, N_EXPERTS), dtype=jnp.bool_)
    gcur = group_scores
    for _ in range(TOPK_GROUP):
        gmax = jnp.max(gcur, axis=1, keepdims=True)
        gsel = jnp.min(jnp.where(gcur == gmax, glane, N_GROUP), axis=1,
                       keepdims=True)
        mask64 = mask64 | (gid == gsel)
        gcur = jnp.where(glane == gsel, neg, gcur)

    masked = jnp.where(mask64, s4c, 0.0)
    idxs, ws = [], []
    cur = masked
    for _ in range(TOP_K):
        vmax = jnp.max(cur, axis=1, keepdims=True)
        sel = jnp.min(jnp.where(cur == vmax, lane, N_EXPERTS), axis=1,
                      keepdims=True)
        idxs.append(sel)
        ws.append(jnp.sum(jnp.where(lane == sel, scores, 0.0), axis=1,
                          keepdims=True))
        cur = jnp.where(lane == sel, neg, cur)
    topk_idx = jnp.concatenate(idxs, axis=1)
    topk_w = jnp.concatenate(ws, axis=1)
    denom = jnp.sum(topk_w, axis=1, keepdims=True) + 1e-20
    idx_ref[...] = topk_idx
    w_ref[...] = topk_w / denom * SCALE


@jax.jit
def kernel(hidden_states, weight, e_score_correction_bias):
    b, s, h = hidden_states.shape
    t = b * s
    hs = hidden_states.reshape(t, h).astype(jnp.float32)
    wt = weight.astype(jnp.float32).T
    bias = e_score_correction_bias.astype(jnp.float32).reshape(1, N_EXPERTS)

    bt = 1024
    grid = (t // bt,)
    logits, idx, w = pl.pallas_call(
        _router_block,
        grid=grid,
        in_specs=[
            pl.BlockSpec((bt, h), lambda i: (i, 0)),
            pl.BlockSpec((h, N_EXPERTS), lambda i: (0, 0)),
            pl.BlockSpec((1, N_EXPERTS), lambda i: (0, 0)),
        ],
        out_specs=[
            pl.BlockSpec((bt, N_EXPERTS), lambda i: (i, 0)),
            pl.BlockSpec((bt, TOP_K), lambda i: (i, 0)),
            pl.BlockSpec((bt, TOP_K), lambda i: (i, 0)),
        ],
        out_shape=[
            jax.ShapeDtypeStruct((t, N_EXPERTS), jnp.float32),
            jax.ShapeDtypeStruct((t, TOP_K), jnp.int32),
            jax.ShapeDtypeStruct((t, TOP_K), jnp.float32),
        ],
    )(hs, wt, bias)
    return idx, w, logits


# speedup vs baseline: 2.1606x; 1.0808x over previous
import jax
import jax.numpy as jnp
from jax.experimental import pallas as pl

TOP_K = 8
N_EXPERTS = 64
N_GROUP = 8
GROUP_SIZE = 8
TOPK_GROUP = 4
SCALE = 2.5


def _router_block(x_ref, wt_ref, bias_ref, logits_ref, idx_ref, w_ref):
    x = x_ref[...]
    wt = wt_ref[...]
    logits = jnp.dot(x, wt, preferred_element_type=jnp.float32)
    logits_ref[...] = logits
    scores = jax.nn.sigmoid(logits)
    s4c = scores + bias_ref[...]

    bt = x.shape[0]
    lane = jax.lax.broadcasted_iota(jnp.int32, (bt, N_EXPERTS), 1)
    gid = lane // GROUP_SIZE
    neg = jnp.float32(-jnp.inf)

    def partner(v, d):
        return jnp.where((lane & d) == 0, jnp.roll(v, -d, axis=1),
                         jnp.roll(v, d, axis=1))

    p = partner(s4c, 1)
    hi = jnp.maximum(s4c, p)
    lo = jnp.minimum(s4c, p)
    for d in (2, 4):
        ph = partner(hi, d)
        plo = jnp.where(hi >= ph, lo, partner(lo, d))
        hi, lo = jnp.maximum(hi, ph), jnp.maximum(jnp.minimum(hi, ph), plo)
    gs = hi + lo

    rank = jnp.zeros((bt, N_EXPERTS), jnp.int32)
    for k in range(1, N_GROUP):
        r = jnp.roll(gs, GROUP_SIZE * k, axis=1)
        beats = (r > gs) | ((r == gs) & (gid >= k))
        rank = rank + beats.astype(jnp.int32)
    mask64 = rank < TOPK_GROUP

    masked = jnp.where(mask64, s4c, 0.0)
    idxs, ws = [], []
    cur = masked
    for _ in range(TOP_K):
        vmax = jnp.max(cur, axis=1, keepdims=True)
        sel = jnp.min(jnp.where(cur == vmax, lane, N_EXPERTS), axis=1,
                      keepdims=True)
        idxs.append(sel)
        ws.append(jnp.sum(jnp.where(lane == sel, scores, 0.0), axis=1,
                          keepdims=True))
        cur = jnp.where(lane == sel, neg, cur)
    topk_idx = jnp.concatenate(idxs, axis=1)
    topk_w = jnp.concatenate(ws, axis=1)
    denom = jnp.sum(topk_w, axis=1, keepdims=True) + 1e-20
    idx_ref[...] = topk_idx
    w_ref[...] = topk_w / denom * SCALE


@jax.jit
def kernel(hidden_states, weight, e_score_correction_bias):
    b, s, h = hidden_states.shape
    t = b * s
    hs = hidden_states.reshape(t, h).astype(jnp.float32)
    wt = weight.astype(jnp.float32).T
    bias = e_score_correction_bias.astype(jnp.float32).reshape(1, N_EXPERTS)

    bt = 1024
    grid = (t // bt,)
    logits, idx, w = pl.pallas_call(
        _router_block,
        grid=grid,
        in_specs=[
            pl.BlockSpec((bt, h), lambda i: (i, 0)),
            pl.BlockSpec((h, N_EXPERTS), lambda i: (0, 0)),
            pl.BlockSpec((1, N_EXPERTS), lambda i: (0, 0)),
        ],
        out_specs=[
            pl.BlockSpec((bt, N_EXPERTS), lambda i: (i, 0)),
            pl.BlockSpec((bt, TOP_K), lambda i: (i, 0)),
            pl.BlockSpec((bt, TOP_K), lambda i: (i, 0)),
        ],
        out_shape=[
            jax.ShapeDtypeStruct((t, N_EXPERTS), jnp.float32),
            jax.ShapeDtypeStruct((t, TOP_K), jnp.int32),
            jax.ShapeDtypeStruct((t, TOP_K), jnp.float32),
        ],
    )(hs, wt, bias)
    return idx, w, logits
